# Initial kernel scaffold; baseline (speedup 1.0000x reference)
import jax
import jax.numpy as jnp
from jax.experimental import pallas as pl


def kernel(x, emb, W2, b2, W3, b3, W4):
    raise NotImplementedError("write your pallas kernel here")


# speedup vs baseline: 4.5350x; 4.5350x over previous
import functools

import jax
import jax.numpy as jnp
from jax import lax
from jax.experimental import pallas as pl
from jax.experimental.pallas import tpu as pltpu
from jax.experimental.pallas import tpu_sc as plsc

B = 16384
K = 32
KU = 29
D = 128
V = 7424
NC, NS = 2, 16
NW = NC * NS
EPW = B // NW
IDX_ROWS = EPW * K // 128
EPC = 128 // K
HS_ROWS = 128
STEPS_PER_FLUSH = HS_ROWS // EPC


def _sc_gather_sum(xr, emb):
    mesh = plsc.VectorSubcoreMesh(core_axis_name="c", subcore_axis_name="s")

    @functools.partial(
        pl.kernel,
        out_type=jax.ShapeDtypeStruct((B, D), jnp.float32),
        mesh=mesh,
        scratch_types=[
            pltpu.VMEM((IDX_ROWS, 128), jnp.int32),
            pltpu.VMEM((2, 128, D), jnp.float32),
            pltpu.VMEM((HS_ROWS, D), jnp.float32),
            pltpu.SemaphoreType.DMA,
            pltpu.SemaphoreType.DMA,
        ],
    )
    def k(xr_hbm, emb_hbm, out_hbm, idx_v, buf_v, hs_v, sem0, sem1):
        wid = lax.axis_index("s") * NC + lax.axis_index("c")
        pltpu.sync_copy(xr_hbm.at[pl.ds(wid * IDX_ROWS, IDX_ROWS)], idx_v)
        sems = (sem0, sem1)
        pltpu.async_copy(emb_hbm.at[idx_v.at[0]], buf_v.at[0], sem0)
        pltpu.async_copy(emb_hbm.at[idx_v.at[1]], buf_v.at[1], sem1)

        def step(jj, _):
            for b in range(2):
                j = 2 * jj + b
                buf = buf_v.at[b]
                pltpu.make_async_copy(emb_hbm.at[idx_v.at[j]], buf, sems[b]).wait()
                row0 = (j % STEPS_PER_FLUSH) * EPC
                for e in range(EPC):
                    for cg in range(D // 16):
                        sl = pl.ds(cg * 16, 16)
                        acc = buf[e * K, sl]
                        for r in range(1, KU):
                            acc = acc + buf[e * K + r, sl]
                        hs_v[row0 + e, sl] = acc

                @pl.when(j + 2 < IDX_ROWS)
                def _():
                    pltpu.async_copy(emb_hbm.at[idx_v.at[j + 2]], buf, sems[b])

                @pl.when((b == 1) & (jj % (STEPS_PER_FLUSH // 2) == STEPS_PER_FLUSH // 2 - 1))
                def _():
                    blk = jj // (STEPS_PER_FLUSH // 2)
                    pltpu.sync_copy(
                        hs_v, out_hbm.at[pl.ds(wid * EPW + blk * HS_ROWS, HS_ROWS)]
                    )
            return _

        lax.fori_loop(0, IDX_ROWS // 2, step, None)

    return k(xr, emb)


def _mlp_body(h_ref, w2t_ref, b2_ref, w3a_ref, w3b_ref, b3_ref, w4a_ref, w4b_ref, o_ref):
    h = jnp.maximum(h_ref[...], 0.0)
    l1 = jnp.dot(h, w2t_ref[...], preferred_element_type=jnp.float32) + b2_ref[...]
    l2 = (
        jnp.dot(jnp.maximum(l1, 0.0), w3a_ref[...], preferred_element_type=jnp.float32)
        + jnp.dot(jnp.maximum(-l1, 0.0), w3b_ref[...], preferred_element_type=jnp.float32)
        + b3_ref[...]
    )
    o_ref[...] = jnp.dot(
        jnp.maximum(l2, 0.0), w4a_ref[...], preferred_element_type=jnp.float32
    ) + jnp.dot(jnp.maximum(-l2, 0.0), w4b_ref[...], preferred_element_type=jnp.float32)


def _mlp(h, W2, b2, W3, b3, W4):
    BB = 1024
    w2t = W2.T
    w3a = W3[:, :32].T
    w3b = W3[:, 32:].T
    w4a = W4[:, :32].T
    w4b = W4[:, 32:].T
    b2r = b2.reshape(1, 32)
    b3r = b3.reshape(1, 32)
    full = lambda s: pl.BlockSpec(s, lambda i: (0, 0))
    return pl.pallas_call(
        _mlp_body,
        grid=(B // BB,),
        in_specs=[
            pl.BlockSpec((BB, D), lambda i: (i, 0)),
            full((D, 32)),
            full((1, 32)),
            full((32, 32)),
            full((32, 32)),
            full((1, 32)),
            full((32, 1)),
            full((32, 1)),
        ],
        out_specs=pl.BlockSpec((BB, 1), lambda i: (i, 0)),
        out_shape=jax.ShapeDtypeStruct((B, 1), jnp.float32),
    )(h, w2t, b2r, w3a, w3b, b3r, w4a, w4b)


def kernel(x, emb, W2, b2, W3, b3, W4):
    xr = x.reshape(B * K // 128, 128)
    hsum = _sc_gather_sum(xr, emb)
    return _mlp(hsum, W2, b2, W3, b3, W4)


# speedup vs baseline: 8.1904x; 1.8061x over previous
import functools

import jax
import jax.numpy as jnp
from jax import lax
from jax.experimental import pallas as pl
from jax.experimental.pallas import tpu as pltpu
from jax.experimental.pallas import tpu_sc as plsc

B = 16384
K = 32
KU = 29
D = 128
V = 7424
NC, NS = 2, 16
NW = NC * NS
EPW = B // NW
IDX_ROWS = EPW * K // 128
EPC = 128 // K
HS_ROWS = 128
STEPS_PER_FLUSH = HS_ROWS // EPC


def _sc_gather_sum(xr, emb):
    mesh = plsc.VectorSubcoreMesh(core_axis_name="c", subcore_axis_name="s")

    @functools.partial(
        pl.kernel,
        out_type=jax.ShapeDtypeStruct((B, D), jnp.float32),
        mesh=mesh,
        scratch_types=[
            pltpu.VMEM((IDX_ROWS, 128), jnp.int32),
            pltpu.VMEM((2, 128, D), jnp.float32),
            pltpu.VMEM((2, HS_ROWS, D), jnp.float32),
            pltpu.SemaphoreType.DMA,
            pltpu.SemaphoreType.DMA,
            pltpu.SemaphoreType.DMA,
        ],
    )
    def k(xr_hbm, emb_hbm, out_hbm, idx_v, buf_v, hs_v, sem0, sem1, sem_out):
        wid = lax.axis_index("s") * NC + lax.axis_index("c")
        pltpu.sync_copy(xr_hbm.at[pl.ds(wid * IDX_ROWS, IDX_ROWS)], idx_v)
        sems = (sem0, sem1)
        pltpu.async_copy(emb_hbm.at[idx_v.at[0]], buf_v.at[0], sem0)
        pltpu.async_copy(emb_hbm.at[idx_v.at[1]], buf_v.at[1], sem1)
        NCG = D // 16

        def step(jj, _):
            for b in range(2):
                j = 2 * jj + b
                buf = buf_v.at[b]
                pltpu.make_async_copy(emb_hbm.at[idx_v.at[j]], buf, sems[b]).wait()
                jmod = j % STEPS_PER_FLUSH
                hs = hs_v.at[(j // STEPS_PER_FLUSH) % 2]
                for e in range(EPC):
                    for half in range(2):
                        cgs = range(half * NCG // 2, (half + 1) * NCG // 2)
                        accs = {cg: buf[e * K, pl.ds(cg * 16, 16)] for cg in cgs}
                        for r in range(1, KU):
                            for cg in cgs:
                                accs[cg] = accs[cg] + buf[e * K + r, pl.ds(cg * 16, 16)]
                        for cg in cgs:
                            hs[jmod * EPC + e, pl.ds(cg * 16, 16)] = accs[cg]

                @pl.when(j + 2 < IDX_ROWS)
                def _():
                    pltpu.async_copy(emb_hbm.at[idx_v.at[j + 2]], buf, sems[b])

                @pl.when((b == 1) & (jj % (STEPS_PER_FLUSH // 2) == STEPS_PER_FLUSH // 2 - 1))
                def _():
                    blk = jj // (STEPS_PER_FLUSH // 2)

                    @pl.when(blk >= 1)
                    def _():
                        pltpu.make_async_copy(
                            hs_v.at[0], out_hbm.at[pl.ds(0, HS_ROWS)], sem_out
                        ).wait()

                    pltpu.async_copy(
                        hs_v.at[blk % 2],
                        out_hbm.at[pl.ds(wid * EPW + blk * HS_ROWS, HS_ROWS)],
                        sem_out,
                    )
            return _

        lax.fori_loop(0, IDX_ROWS // 2, step, None)
        pltpu.make_async_copy(hs_v.at[0], out_hbm.at[pl.ds(0, HS_ROWS)], sem_out).wait()

    return k(xr, emb)


def _mlp_body(h_ref, w2t_ref, b2_ref, w3a_ref, w3b_ref, b3_ref, w4a_ref, w4b_ref, o_ref):
    h = jnp.maximum(h_ref[...], 0.0)
    l1 = jnp.dot(h, w2t_ref[...], preferred_element_type=jnp.float32) + b2_ref[...]
    l2 = (
        jnp.dot(jnp.maximum(l1, 0.0), w3a_ref[...], preferred_element_type=jnp.float32)
        + jnp.dot(jnp.maximum(-l1, 0.0), w3b_ref[...], preferred_element_type=jnp.float32)
        + b3_ref[...]
    )
    o_ref[...] = jnp.dot(
        jnp.maximum(l2, 0.0), w4a_ref[...], preferred_element_type=jnp.float32
    ) + jnp.dot(jnp.maximum(-l2, 0.0), w4b_ref[...], preferred_element_type=jnp.float32)


def _mlp(h, W2, b2, W3, b3, W4):
    BB = 1024
    w2t = W2.T
    w3a = W3[:, :32].T
    w3b = W3[:, 32:].T
    w4a = W4[:, :32].T
    w4b = W4[:, 32:].T
    b2r = b2.reshape(1, 32)
    b3r = b3.reshape(1, 32)
    full = lambda s: pl.BlockSpec(s, lambda i: (0, 0))
    return pl.pallas_call(
        _mlp_body,
        grid=(B // BB,),
        in_specs=[
            pl.BlockSpec((BB, D), lambda i: (i, 0)),
            full((D, 32)),
            full((1, 32)),
            full((32, 32)),
            full((32, 32)),
            full((1, 32)),
            full((32, 1)),
            full((32, 1)),
        ],
        out_specs=pl.BlockSpec((BB, 1), lambda i: (i, 0)),
        out_shape=jax.ShapeDtypeStruct((B, 1), jnp.float32),
    )(h, w2t, b2r, w3a, w3b, b3r, w4a, w4b)


def kernel(x, emb, W2, b2, W3, b3, W4):
    xr = x.reshape(B * K // 128, 128)
    hsum = _sc_gather_sum(xr, emb)
    return _mlp(hsum, W2, b2, W3, b3, W4)
